# Initial kernel scaffold; baseline (speedup 1.0000x reference)
import jax
import jax.numpy as jnp
from jax.experimental import pallas as pl


def kernel(x, lengths, table, W1, b1, W2, b2):
    raise NotImplementedError("write your pallas kernel here")


# speedup vs baseline: 7.4994x; 7.4994x over previous
import functools

import jax
import jax.numpy as jnp
from jax import lax
from jax.experimental import pallas as pl
from jax.experimental.pallas import tpu as pltpu
from jax.experimental.pallas import tpu_sc as plsc

B, L, V, D, H, C = 4096, 200, 100000, 128, 32, 10

NC, NS, LANES = 2, 16, 16
NW = NC * NS
BPW = B // NW
NV = D // LANES

CH0, CH1 = 104, 96


def _pool_body(x_hbm, table_hbm, out_sum_hbm, out_max_hbm,
               idx_v, rows_v, osum_v, omax_v, sem):
    cid = lax.axis_index("c")
    sid = lax.axis_index("s")
    wid = sid * NC + cid
    base = wid * BPW

    pltpu.sync_copy(x_hbm.at[pl.ds(pl.multiple_of(base * L, 8), BPW * L)],
                    idx_v)

    def start_gather(s):
        off = pl.multiple_of(s * L, 8)
        c0 = pltpu.async_copy(table_hbm.at[idx_v.at[pl.ds(off, CH0)]],
                              rows_v.at[pl.ds(0, CH0)], sem)
        c1 = pltpu.async_copy(table_hbm.at[idx_v.at[pl.ds(off + CH0, CH1)]],
                              rows_v.at[pl.ds(CH0, CH1)], sem)
        return c0, c1

    def pool_rows():
        def row_body(r, carry):
            ss = list(carry[:NV])
            mm = list(carry[NV:])
            for c in range(NV):
                v = rows_v[r, pl.ds(c * LANES, LANES)]
                ss[c] = ss[c] + v
                mm[c] = jnp.maximum(mm[c], v)
            return tuple(ss) + tuple(mm)

        init = (tuple(jnp.zeros((LANES,), jnp.float32) for _ in range(NV)) +
                tuple(jnp.full((LANES,), -jnp.inf, jnp.float32)
                      for _ in range(NV)))
        return lax.fori_loop(0, L, row_body, init)

    def sample_body(s, _):
        c0, c1 = start_gather(s)
        c0.wait()
        c1.wait()
        res = pool_rows()
        for c in range(NV):
            osum_v[s, pl.ds(c * LANES, LANES)] = res[c]
            omax_v[s, pl.ds(c * LANES, LANES)] = res[NV + c]
        return 0

    lax.fori_loop(0, BPW, sample_body, 0)

    pltpu.sync_copy(osum_v, out_sum_hbm.at[pl.ds(base, BPW)])
    pltpu.sync_copy(omax_v, out_max_hbm.at[pl.ds(base, BPW)])


@jax.jit
def _pool(x_flat, table):
    mesh = plsc.VectorSubcoreMesh(core_axis_name="c", subcore_axis_name="s",
                                  num_cores=NC, num_subcores=NS)
    return pl.kernel(
        _pool_body,
        out_type=(jax.ShapeDtypeStruct((B, D), jnp.float32),
                  jax.ShapeDtypeStruct((B, D), jnp.float32)),
        mesh=mesh,
        scratch_types=[
            pltpu.VMEM((BPW * L,), jnp.int32),
            pltpu.VMEM((L, D), jnp.float32),
            pltpu.VMEM((BPW, D), jnp.float32),
            pltpu.VMEM((BPW, D), jnp.float32),
            pltpu.SemaphoreType.DMA,
        ],
    )(x_flat, table)


def _mlp_body(sum_ref, max_ref, len_ref, w1a_ref, w1b_ref, b1_ref,
              w2_ref, b2_ref, out_ref):
    mean = sum_ref[...] / len_ref[...]
    h = jnp.dot(mean, w1a_ref[...], preferred_element_type=jnp.float32)
    h = h + jnp.dot(max_ref[...], w1b_ref[...],
                    preferred_element_type=jnp.float32)
    h = jnp.maximum(h + b1_ref[...], 0.0)
    out_ref[...] = (jnp.dot(h, w2_ref[...],
                            preferred_element_type=jnp.float32) + b2_ref[...])


@jax.jit
def _mlp(sums, maxs, len_col, w1a, w1b, b1r, w2t, b2r):
    return pl.pallas_call(
        _mlp_body,
        out_shape=jax.ShapeDtypeStruct((B, C), jnp.float32),
    )(sums, maxs, len_col, w1a, w1b, b1r, w2t, b2r)


def kernel(x, lengths, table, W1, b1, W2, b2):
    sums, maxs = _pool(x.reshape(-1), table)
    return _mlp(sums, maxs,
                lengths.astype(jnp.float32).reshape(B, 1),
                W1[:, :D].T, W1[:, D:].T, b1.reshape(1, H),
                W2.T, b2.reshape(1, C))


# speedup vs baseline: 13.0271x; 1.7371x over previous
import functools

import jax
import jax.numpy as jnp
from jax import lax
from jax.experimental import pallas as pl
from jax.experimental.pallas import tpu as pltpu
from jax.experimental.pallas import tpu_sc as plsc

B, L, V, D, H, C = 4096, 200, 100000, 128, 32, 10

NC, NS, LANES = 2, 16, 16
NW = NC * NS
BPW = B // NW
NV = D // LANES

CH0, CH1 = 104, 96


def _pool_body(x_hbm, table_hbm, out_sum_hbm, out_max_hbm,
               idx_v, rows0_v, rows1_v, osum_v, omax_v, sem0, sem1):
    cid = lax.axis_index("c")
    sid = lax.axis_index("s")
    wid = sid * NC + cid
    base = wid * BPW

    pltpu.sync_copy(x_hbm.at[pl.ds(pl.multiple_of(base * L, 8), BPW * L)],
                    idx_v)

    def start_gather(s, rows_v, sem):
        off = pl.multiple_of(s * L, 8)
        pltpu.async_copy(table_hbm.at[idx_v.at[pl.ds(off, CH0)]],
                         rows_v.at[pl.ds(0, CH0)], sem)
        pltpu.async_copy(table_hbm.at[idx_v.at[pl.ds(off + CH0, CH1)]],
                         rows_v.at[pl.ds(CH0, CH1)], sem)

    def wait_gather(rows_v, sem):
        pltpu.make_async_copy(table_hbm.at[idx_v.at[pl.ds(0, CH0)]],
                              rows_v.at[pl.ds(0, CH0)], sem).wait()
        pltpu.make_async_copy(table_hbm.at[idx_v.at[pl.ds(CH0, CH1)]],
                              rows_v.at[pl.ds(CH0, CH1)], sem).wait()

    def pool_rows(rows_v):
        def row_body(r, carry):
            ss = list(carry[:NV])
            mm = list(carry[NV:])
            for c in range(NV):
                v = rows_v[r, pl.ds(c * LANES, LANES)]
                ss[c] = ss[c] + v
                mm[c] = jnp.maximum(mm[c], v)
            return tuple(ss) + tuple(mm)

        init = (tuple(jnp.zeros((LANES,), jnp.float32) for _ in range(NV)) +
                tuple(jnp.full((LANES,), -jnp.inf, jnp.float32)
                      for _ in range(NV)))
        return lax.fori_loop(0, L, row_body, init)

    def store(s, res):
        for c in range(NV):
            osum_v[s, pl.ds(c * LANES, LANES)] = res[c]
            omax_v[s, pl.ds(c * LANES, LANES)] = res[NV + c]

    npair = BPW // 2
    start_gather(0, rows0_v, sem0)
    start_gather(1, rows1_v, sem1)

    def pair_body(p, _):
        s0 = 2 * p
        wait_gather(rows0_v, sem0)
        res0 = pool_rows(rows0_v)

        @pl.when(p < npair - 1)
        def _():
            start_gather(s0 + 2, rows0_v, sem0)

        store(s0, res0)
        wait_gather(rows1_v, sem1)
        res1 = pool_rows(rows1_v)

        @pl.when(p < npair - 1)
        def _():
            start_gather(s0 + 3, rows1_v, sem1)

        store(s0 + 1, res1)
        return 0

    lax.fori_loop(0, npair, pair_body, 0)

    pltpu.sync_copy(osum_v, out_sum_hbm.at[pl.ds(base, BPW)])
    pltpu.sync_copy(omax_v, out_max_hbm.at[pl.ds(base, BPW)])


@jax.jit
def _pool(x_flat, table):
    mesh = plsc.VectorSubcoreMesh(core_axis_name="c", subcore_axis_name="s",
                                  num_cores=NC, num_subcores=NS)
    return pl.kernel(
        _pool_body,
        out_type=(jax.ShapeDtypeStruct((B, D), jnp.float32),
                  jax.ShapeDtypeStruct((B, D), jnp.float32)),
        mesh=mesh,
        scratch_types=[
            pltpu.VMEM((BPW * L,), jnp.int32),
            pltpu.VMEM((L, D), jnp.float32),
            pltpu.VMEM((L, D), jnp.float32),
            pltpu.VMEM((BPW, D), jnp.float32),
            pltpu.VMEM((BPW, D), jnp.float32),
            pltpu.SemaphoreType.DMA,
            pltpu.SemaphoreType.DMA,
        ],
    )(x_flat, table)


def _mlp_body(sum_ref, max_ref, len_ref, w1a_ref, w1b_ref, b1_ref,
              w2_ref, b2_ref, out_ref):
    mean = sum_ref[...] / len_ref[...]
    h = jnp.dot(mean, w1a_ref[...], preferred_element_type=jnp.float32)
    h = h + jnp.dot(max_ref[...], w1b_ref[...],
                    preferred_element_type=jnp.float32)
    h = jnp.maximum(h + b1_ref[...], 0.0)
    out_ref[...] = (jnp.dot(h, w2_ref[...],
                            preferred_element_type=jnp.float32) + b2_ref[...])


@jax.jit
def _mlp(sums, maxs, len_col, w1a, w1b, b1r, w2t, b2r):
    return pl.pallas_call(
        _mlp_body,
        out_shape=jax.ShapeDtypeStruct((B, C), jnp.float32),
    )(sums, maxs, len_col, w1a, w1b, b1r, w2t, b2r)


def kernel(x, lengths, table, W1, b1, W2, b2):
    sums, maxs = _pool(x.reshape(-1), table)
    return _mlp(sums, maxs,
                lengths.astype(jnp.float32).reshape(B, 1),
                W1[:, :D].T, W1[:, D:].T, b1.reshape(1, H),
                W2.T, b2.reshape(1, C))


# speedup vs baseline: 13.0427x; 1.0012x over previous
import functools

import jax
import jax.numpy as jnp
from jax import lax
from jax.experimental import pallas as pl
from jax.experimental.pallas import tpu as pltpu
from jax.experimental.pallas import tpu_sc as plsc

B, L, V, D, H, C = 4096, 200, 100000, 128, 32, 10

NC, NS, LANES = 2, 16, 16
NW = NC * NS
BPW = B // NW
NV = D // LANES

CH0, CH1 = 104, 96


def _pool_body(x_hbm, table_hbm, out_sum_hbm, out_max_hbm,
               idx_v, rows0_v, rows1_v, osum_v, omax_v, sem0, sem1):
    cid = lax.axis_index("c")
    sid = lax.axis_index("s")
    wid = sid * NC + cid
    base = wid * BPW

    pltpu.sync_copy(x_hbm.at[pl.ds(pl.multiple_of(base * L, 8), BPW * L)],
                    idx_v)

    def start_gather(s, rows_v, sem):
        off = pl.multiple_of(s * L, 8)
        pltpu.async_copy(table_hbm.at[idx_v.at[pl.ds(off, CH0)]],
                         rows_v.at[pl.ds(0, CH0)], sem)
        pltpu.async_copy(table_hbm.at[idx_v.at[pl.ds(off + CH0, CH1)]],
                         rows_v.at[pl.ds(CH0, CH1)], sem)

    def wait_gather(rows_v, sem):
        pltpu.make_async_copy(table_hbm.at[idx_v.at[pl.ds(0, CH0)]],
                              rows_v.at[pl.ds(0, CH0)], sem).wait()
        pltpu.make_async_copy(table_hbm.at[idx_v.at[pl.ds(CH0, CH1)]],
                              rows_v.at[pl.ds(CH0, CH1)], sem).wait()

    def pool_rows(rows_v):
        def row_body(r, carry):
            ss = list(carry[:NV])
            mm = list(carry[NV:])
            for c in range(NV):
                v = rows_v[r, pl.ds(c * LANES, LANES)]
                ss[c] = ss[c] + v
                mm[c] = jnp.maximum(mm[c], v)
            return tuple(ss) + tuple(mm)

        init = (tuple(jnp.zeros((LANES,), jnp.float32) for _ in range(NV)) +
                tuple(jnp.full((LANES,), -jnp.inf, jnp.float32)
                      for _ in range(NV)))
        return lax.fori_loop(0, L, row_body, init, unroll=8)

    def store(s, res):
        for c in range(NV):
            osum_v[s, pl.ds(c * LANES, LANES)] = res[c]
            omax_v[s, pl.ds(c * LANES, LANES)] = res[NV + c]

    npair = BPW // 2
    start_gather(0, rows0_v, sem0)
    start_gather(1, rows1_v, sem1)

    def pair_body(p, _):
        s0 = 2 * p
        wait_gather(rows0_v, sem0)
        res0 = pool_rows(rows0_v)

        @pl.when(p < npair - 1)
        def _():
            start_gather(s0 + 2, rows0_v, sem0)

        store(s0, res0)
        wait_gather(rows1_v, sem1)
        res1 = pool_rows(rows1_v)

        @pl.when(p < npair - 1)
        def _():
            start_gather(s0 + 3, rows1_v, sem1)

        store(s0 + 1, res1)
        return 0

    lax.fori_loop(0, npair, pair_body, 0)

    pltpu.sync_copy(osum_v, out_sum_hbm.at[pl.ds(base, BPW)])
    pltpu.sync_copy(omax_v, out_max_hbm.at[pl.ds(base, BPW)])


@jax.jit
def _pool(x_flat, table):
    mesh = plsc.VectorSubcoreMesh(core_axis_name="c", subcore_axis_name="s",
                                  num_cores=NC, num_subcores=NS)
    return pl.kernel(
        _pool_body,
        out_type=(jax.ShapeDtypeStruct((B, D), jnp.float32),
                  jax.ShapeDtypeStruct((B, D), jnp.float32)),
        mesh=mesh,
        scratch_types=[
            pltpu.VMEM((BPW * L,), jnp.int32),
            pltpu.VMEM((L, D), jnp.float32),
            pltpu.VMEM((L, D), jnp.float32),
            pltpu.VMEM((BPW, D), jnp.float32),
            pltpu.VMEM((BPW, D), jnp.float32),
            pltpu.SemaphoreType.DMA,
            pltpu.SemaphoreType.DMA,
        ],
    )(x_flat, table)


def _mlp_body(sum_ref, max_ref, len_ref, w1a_ref, w1b_ref, b1_ref,
              w2_ref, b2_ref, out_ref):
    mean = sum_ref[...] / len_ref[...]
    h = jnp.dot(mean, w1a_ref[...], preferred_element_type=jnp.float32)
    h = h + jnp.dot(max_ref[...], w1b_ref[...],
                    preferred_element_type=jnp.float32)
    h = jnp.maximum(h + b1_ref[...], 0.0)
    out_ref[...] = (jnp.dot(h, w2_ref[...],
                            preferred_element_type=jnp.float32) + b2_ref[...])


@jax.jit
def _mlp(sums, maxs, len_col, w1a, w1b, b1r, w2t, b2r):
    return pl.pallas_call(
        _mlp_body,
        out_shape=jax.ShapeDtypeStruct((B, C), jnp.float32),
    )(sums, maxs, len_col, w1a, w1b, b1r, w2t, b2r)


def kernel(x, lengths, table, W1, b1, W2, b2):
    sums, maxs = _pool(x.reshape(-1), table)
    return _mlp(sums, maxs,
                lengths.astype(jnp.float32).reshape(B, 1),
                W1[:, :D].T, W1[:, D:].T, b1.reshape(1, H),
                W2.T, b2.reshape(1, C))


# speedup vs baseline: 13.0556x; 1.0010x over previous
import jax
import jax.numpy as jnp
from jax import lax
from jax.experimental import pallas as pl
from jax.experimental.pallas import tpu as pltpu
from jax.experimental.pallas import tpu_sc as plsc

B, L, V, D, H, C = 4096, 200, 100000, 128, 32, 10

NC, NS, LANES = 2, 16, 16
NW = NC * NS
BPW = B // NW
NV = D // LANES

CH0, CH1 = 104, 96


def _pool_body(x_hbm, table_hbm, out_sum_hbm, out_max_hbm,
               idx_v, rows0_v, rows1_v, osum_v, omax_v, sem0, sem1):
    cid = lax.axis_index("c")
    sid = lax.axis_index("s")
    wid = sid * NC + cid
    base = wid * BPW

    pltpu.sync_copy(x_hbm.at[pl.ds(pl.multiple_of(base * L, 8), BPW * L)],
                    idx_v)

    def start_gather(s, rows_v, sem):
        off = pl.multiple_of(s * L, 8)
        pltpu.async_copy(table_hbm.at[idx_v.at[pl.ds(off, CH0)]],
                         rows_v.at[pl.ds(0, CH0)], sem)
        pltpu.async_copy(table_hbm.at[idx_v.at[pl.ds(off + CH0, CH1)]],
                         rows_v.at[pl.ds(CH0, CH1)], sem)

    def wait_gather(rows_v, sem):
        pltpu.make_async_copy(table_hbm.at[idx_v.at[pl.ds(0, CH0)]],
                              rows_v.at[pl.ds(0, CH0)], sem).wait()
        pltpu.make_async_copy(table_hbm.at[idx_v.at[pl.ds(CH0, CH1)]],
                              rows_v.at[pl.ds(CH0, CH1)], sem).wait()

    def pool_rows(rows_v):
        def row_body(r, carry):
            ss = list(carry[:NV])
            mm = list(carry[NV:])
            for c in range(NV):
                v = rows_v[r, pl.ds(c * LANES, LANES)]
                ss[c] = ss[c] + v
                mm[c] = jnp.maximum(mm[c], v)
            return tuple(ss) + tuple(mm)

        init = (tuple(jnp.zeros((LANES,), jnp.float32) for _ in range(NV)) +
                tuple(jnp.full((LANES,), -jnp.inf, jnp.float32)
                      for _ in range(NV)))
        return lax.fori_loop(0, L, row_body, init)

    def store(s, res):
        for c in range(NV):
            osum_v[s, pl.ds(c * LANES, LANES)] = res[c]
            omax_v[s, pl.ds(c * LANES, LANES)] = res[NV + c]

    npair = BPW // 2
    start_gather(0, rows0_v, sem0)
    start_gather(1, rows1_v, sem1)

    def pair_body(p, _):
        s0 = 2 * p
        wait_gather(rows0_v, sem0)
        res0 = pool_rows(rows0_v)

        @pl.when(p < npair - 1)
        def _():
            start_gather(s0 + 2, rows0_v, sem0)

        store(s0, res0)
        wait_gather(rows1_v, sem1)
        res1 = pool_rows(rows1_v)

        @pl.when(p < npair - 1)
        def _():
            start_gather(s0 + 3, rows1_v, sem1)

        store(s0 + 1, res1)
        return 0

    lax.fori_loop(0, npair, pair_body, 0)

    pltpu.sync_copy(osum_v, out_sum_hbm.at[pl.ds(base, BPW)])
    pltpu.sync_copy(omax_v, out_max_hbm.at[pl.ds(base, BPW)])


@jax.jit
def _pool(x_flat, table):
    mesh = plsc.VectorSubcoreMesh(core_axis_name="c", subcore_axis_name="s",
                                  num_cores=NC, num_subcores=NS)
    return pl.kernel(
        _pool_body,
        out_type=(jax.ShapeDtypeStruct((B, D), jnp.float32),
                  jax.ShapeDtypeStruct((B, D), jnp.float32)),
        mesh=mesh,
        scratch_types=[
            pltpu.VMEM((BPW * L,), jnp.int32),
            pltpu.VMEM((L, D), jnp.float32),
            pltpu.VMEM((L, D), jnp.float32),
            pltpu.VMEM((BPW, D), jnp.float32),
            pltpu.VMEM((BPW, D), jnp.float32),
            pltpu.SemaphoreType.DMA,
            pltpu.SemaphoreType.DMA,
        ],
    )(x_flat, table)


def _mlp_body(sum_ref, max_ref, len_ref, w1_ref, b1_ref, w2_ref, b2_ref,
              out_ref):
    dn = (((1,), (1,)), ((), ()))
    mean = sum_ref[...] / len_ref[...]
    h = lax.dot_general(mean, w1_ref[:, :D], dn,
                        preferred_element_type=jnp.float32)
    h = h + lax.dot_general(max_ref[...], w1_ref[:, D:], dn,
                            preferred_element_type=jnp.float32)
    h = jnp.maximum(h + b1_ref[...], 0.0)
    out_ref[...] = lax.dot_general(h, w2_ref[...], dn,
                                   preferred_element_type=jnp.float32) + b2_ref[...]


@jax.jit
def _mlp(sums, maxs, len_col, W1, b1r, W2, b2r):
    return pl.pallas_call(
        _mlp_body,
        out_shape=jax.ShapeDtypeStruct((B, C), jnp.float32),
    )(sums, maxs, len_col, W1, b1r, W2, b2r)


def kernel(x, lengths, table, W1, b1, W2, b2):
    sums, maxs = _pool(x.reshape(-1), table)
    return _mlp(sums, maxs, lengths.astype(jnp.float32).reshape(B, 1),
                W1, b1.reshape(1, H), W2, b2.reshape(1, C))
